# Initial kernel scaffold; baseline (speedup 1.0000x reference)
import jax
import jax.numpy as jnp
from jax.experimental import pallas as pl


def kernel(xyz, points, W0, gamma0, beta0, W1, gamma1, beta1, W2, gamma2, beta2):
    raise NotImplementedError("write your pallas kernel here")


# speedup vs baseline: 8.8263x; 8.8263x over previous
import functools

import jax
import jax.numpy as jnp
from jax import lax
from jax.experimental import pallas as pl
from jax.experimental.pallas import tpu as pltpu
from jax.experimental.pallas import tpu_sc as plsc

B = 8
N = 4096
S = 1024
K = 32
C_IN = 64
CC = 3 + C_IN
CP = 128
EPS = 1e-5
NROWS = B * S * K
NCENT = B * S


def _fps_body(x_ref, y_ref, z_ref, nx_ref, ny_ref, nz_ref, gi_ref):
    x = x_ref[...]
    y = y_ref[...]
    z = z_ref[...]
    lane_n = lax.broadcasted_iota(jnp.int32, (B, N), 1)
    lane_s = lax.broadcasted_iota(jnp.int32, (B, S), 1)
    boff = lax.broadcasted_iota(jnp.int32, (B, S), 0) * N

    cx0 = x[:, 0:1]
    cy0 = y[:, 0:1]
    cz0 = z[:, 0:1]
    zs = jnp.zeros((B, S), jnp.float32)
    nx0 = jnp.where(lane_s == 0, cx0, zs)
    ny0 = jnp.where(lane_s == 0, cy0, zs)
    nz0 = jnp.where(lane_s == 0, cz0, zs)
    gi0 = jnp.where(lane_s == 0, boff, jnp.zeros((B, S), jnp.int32))
    mind0 = jnp.full((B, N), 1e10, jnp.float32)

    def body(i, st):
        cx, cy, cz, mind, nx, ny, nz, gi = st
        dx = x - cx
        dy = y - cy
        dz = z - cz
        d = dx * dx + dy * dy + dz * dz
        mind = jnp.minimum(mind, d)
        m = jnp.max(mind, axis=1, keepdims=True)
        cand = jnp.where(mind == m, lane_n, N)
        idx = jnp.min(cand, axis=1, keepdims=True)
        hit = lane_n == idx
        fz = jnp.zeros((B, N), jnp.float32)
        cx = jnp.sum(jnp.where(hit, x, fz), axis=1, keepdims=True)
        cy = jnp.sum(jnp.where(hit, y, fz), axis=1, keepdims=True)
        cz = jnp.sum(jnp.where(hit, z, fz), axis=1, keepdims=True)
        here = lane_s == i
        nx = jnp.where(here, cx, nx)
        ny = jnp.where(here, cy, ny)
        nz = jnp.where(here, cz, nz)
        gi = jnp.where(here, idx + boff, gi)
        return (cx, cy, cz, mind, nx, ny, nz, gi)

    st = (cx0, cy0, cz0, mind0, nx0, ny0, nz0, gi0)
    st = lax.fori_loop(1, S, body, st)
    _, _, _, _, nx, ny, nz, gi = st
    nx_ref[...] = nx
    ny_ref[...] = ny
    nz_ref[...] = nz
    gi_ref[...] = gi


def _fps(x, y, z):
    return pl.pallas_call(
        _fps_body,
        out_shape=(
            jax.ShapeDtypeStruct((B, S), jnp.float32),
            jax.ShapeDtypeStruct((B, S), jnp.float32),
            jax.ShapeDtypeStruct((B, S), jnp.float32),
            jax.ShapeDtypeStruct((B, S), jnp.int32),
        ),
    )(x, y, z)


SBLK = 256


def _knn_body(nx_ref, ny_ref, nz_ref, x_ref, y_ref, z_ref, out_ref):
    b = pl.program_id(0)
    xs = nx_ref[0]
    ys = ny_ref[0]
    zs = nz_ref[0]
    xr = x_ref[0]
    yr = y_ref[0]
    zr = z_ref[0]
    def _b(v):
        return v.astype(jnp.bfloat16).astype(jnp.float32)

    a2 = xs * xs + ys * ys + zs * zs
    b2 = xr * xr + yr * yr + zr * zr
    ab = _b(xs) * _b(xr) + _b(ys) * _b(yr) + _b(zs) * _b(zr)
    d = (a2 + b2) - 2.0 * ab
    iota = lax.broadcasted_iota(jnp.int32, (SBLK, N), 1)
    kiota = lax.broadcasted_iota(jnp.int32, (SBLK, K), 1)
    boff = b * N

    def body(k, st):
        d, acc = st
        m = jnp.min(d, axis=1, keepdims=True)
        cand = jnp.where(d == m, iota, N)
        idx = jnp.min(cand, axis=1, keepdims=True)
        acc = jnp.where(kiota == k, idx + boff, acc)
        d = jnp.where(iota == idx, jnp.float32(jnp.inf), d)
        return (d, acc)

    _, acc = lax.fori_loop(0, K, body, (d, jnp.zeros((SBLK, K), jnp.int32)))
    out_ref[0] = acc


def _knn(nxc, nyc, nzc, x, y, z):
    grid = (B, S // SBLK)
    x = x[:, None, :]
    y = y[:, None, :]
    z = z[:, None, :]
    cent_spec = pl.BlockSpec((1, SBLK, 1), lambda b, j: (b, j, 0))
    pt_spec = pl.BlockSpec((1, 1, N), lambda b, j: (b, 0, 0))
    return pl.pallas_call(
        _knn_body,
        grid=grid,
        in_specs=[cent_spec, cent_spec, cent_spec, pt_spec, pt_spec, pt_spec],
        out_specs=pl.BlockSpec((1, SBLK, K), lambda b, j: (b, j, 0)),
        out_shape=jax.ShapeDtypeStruct((B, S, K), jnp.int32),
    )(nxc, nyc, nzc, x, y, z)


NW = 32
RPW = NROWS // NW
CPW = RPW // 128
FPW = NCENT // NW
FCW = FPW // 128


def _sc_gather_body(knn_hbm, fps_hbm, tab_hbm, xg_hbm, ng_hbm,
                    idx_v, fidx_v, rows_v, sem):
    wid = lax.axis_index("s") * 2 + lax.axis_index("c")
    pltpu.sync_copy(knn_hbm.at[wid], idx_v)
    pltpu.sync_copy(fps_hbm.at[wid], fidx_v)

    def body(j, _):
        pltpu.async_copy(tab_hbm.at[idx_v.at[j]], rows_v, sem).wait()
        pltpu.sync_copy(rows_v, xg_hbm.at[wid, j])
        return 0

    lax.fori_loop(0, CPW, body, 0)

    def fbody(j, _):
        pltpu.async_copy(tab_hbm.at[fidx_v.at[j]], rows_v, sem).wait()
        pltpu.sync_copy(rows_v, ng_hbm.at[wid, j])
        return 0

    lax.fori_loop(0, FCW, fbody, 0)


def _sc_gather(knn3, fps3, table):
    mesh = plsc.VectorSubcoreMesh(core_axis_name="c", subcore_axis_name="s")
    f = functools.partial(
        pl.kernel,
        mesh=mesh,
        out_type=(
            jax.ShapeDtypeStruct((NW, CPW, 128, CP), jnp.float32),
            jax.ShapeDtypeStruct((NW, FCW, 128, CP), jnp.float32),
        ),
        scratch_types=[
            pltpu.VMEM((CPW, 128), jnp.int32),
            pltpu.VMEM((FCW, 128), jnp.int32),
            pltpu.VMEM((128, CP), jnp.float32),
            pltpu.SemaphoreType.DMA,
        ],
    )(_sc_gather_body)
    return f(knn3, fps3, table)


RBLK = 2048
CBLK = RBLK // K
NBLK = NROWS // RBLK


def _pass1_body(xg_ref, ng_ref, w_ref, y_ref, st_ref, acc):
    i = pl.program_id(0)

    @pl.when(i == 0)
    def _():
        acc[...] = jnp.zeros_like(acc)

    xv = xg_ref[...]
    cm = ng_ref[...]
    col = lax.broadcasted_iota(jnp.int32, (CBLK, CP), 1)
    cz = jnp.where(col < 3, cm, jnp.zeros_like(cm))
    w = w_ref[...]
    y = jnp.dot(xv, w, preferred_element_type=jnp.float32)
    corr = jnp.dot(cz, w, preferred_element_type=jnp.float32)
    y3 = y.reshape(CBLK, K, 64) - corr[:, None, :]
    y2 = y3.reshape(RBLK, 64)
    y_ref[...] = y2
    acc[0:1, :] += jnp.sum(y2, axis=0, keepdims=True)
    acc[1:2, :] += jnp.sum(y2 * y2, axis=0, keepdims=True)

    @pl.when(i == NBLK - 1)
    def _():
        st_ref[...] = acc[...]


def _pass1(xg, ng, w1p):
    return pl.pallas_call(
        _pass1_body,
        grid=(NBLK,),
        in_specs=[
            pl.BlockSpec((RBLK, CP), lambda i: (i, 0)),
            pl.BlockSpec((CBLK, CP), lambda i: (i, 0)),
            pl.BlockSpec((CP, 64), lambda i: (0, 0)),
        ],
        out_specs=(
            pl.BlockSpec((RBLK, 64), lambda i: (i, 0)),
            pl.BlockSpec((8, 64), lambda i: (0, 0)),
        ),
        out_shape=(
            jax.ShapeDtypeStruct((NROWS, 64), jnp.float32),
            jax.ShapeDtypeStruct((8, 64), jnp.float32),
        ),
        scratch_shapes=[pltpu.VMEM((8, 64), jnp.float32)],
    )(xg, ng, w1p)


def _mid_body(cout, y_ref, st_ref, g_ref, b_ref, w_ref, z_ref, so_ref, acc):
    i = pl.program_id(0)

    @pl.when(i == 0)
    def _():
        acc[...] = jnp.zeros_like(acc)

    cnt = jnp.float32(NROWS)
    mean = st_ref[0:1, :] / cnt
    var = st_ref[1:2, :] / cnt - mean * mean
    scale = g_ref[...] / jnp.sqrt(var + EPS)
    shift = b_ref[...] - mean * scale
    y = y_ref[...]
    h = jnp.maximum(y * scale + shift, 0.0)
    z = jnp.dot(h, w_ref[...], preferred_element_type=jnp.float32)
    z_ref[...] = z
    acc[0:1, :] += jnp.sum(z, axis=0, keepdims=True)
    acc[1:2, :] += jnp.sum(z * z, axis=0, keepdims=True)

    @pl.when(i == NBLK - 1)
    def _():
        so_ref[...] = acc[...]


def _mid(y, st, g, b, wt, cout):
    return pl.pallas_call(
        functools.partial(_mid_body, cout),
        grid=(NBLK,),
        in_specs=[
            pl.BlockSpec((RBLK, 64), lambda i: (i, 0)),
            pl.BlockSpec((8, 64), lambda i: (0, 0)),
            pl.BlockSpec((1, 64), lambda i: (0, 0)),
            pl.BlockSpec((1, 64), lambda i: (0, 0)),
            pl.BlockSpec((64, cout), lambda i: (0, 0)),
        ],
        out_specs=(
            pl.BlockSpec((RBLK, cout), lambda i: (i, 0)),
            pl.BlockSpec((8, cout), lambda i: (0, 0)),
        ),
        out_shape=(
            jax.ShapeDtypeStruct((NROWS, cout), jnp.float32),
            jax.ShapeDtypeStruct((8, cout), jnp.float32),
        ),
        scratch_shapes=[pltpu.VMEM((8, cout), jnp.float32)],
    )(y, st, g, b, wt)


def _pool_body(y_ref, st_ref, g_ref, b_ref, o_ref):
    i = pl.program_id(0)
    cnt = jnp.float32(NROWS)
    mean = st_ref[0:1, :] / cnt
    var = st_ref[1:2, :] / cnt - mean * mean
    sc = g_ref[...] / jnp.sqrt(var + EPS)
    scale = sc.reshape(1, 1, 128)
    shift = (b_ref[...] - mean * sc).reshape(1, 1, 128)
    y = y_ref[...]
    h = jnp.maximum(y * scale + shift, 0.0)
    pooled = jnp.max(h, axis=1)
    o_ref[0, :, pl.ds((i % 8) * 128, 128)] = pooled.T


def _pool(y3d, st, g, b):
    return pl.pallas_call(
        _pool_body,
        grid=(NCENT // 128,),
        in_specs=[
            pl.BlockSpec((128, K, 128), lambda i: (i, 0, 0)),
            pl.BlockSpec((8, 128), lambda i: (0, 0)),
            pl.BlockSpec((1, 128), lambda i: (0, 0)),
            pl.BlockSpec((1, 128), lambda i: (0, 0)),
        ],
        out_specs=pl.BlockSpec((1, 128, S), lambda i: (i // 8, 0, 0)),
        out_shape=jax.ShapeDtypeStruct((B, 128, S), jnp.float32),
    )(y3d, st, g, b)


def kernel(xyz, points, W0, gamma0, beta0, W1, gamma1, beta1, W2, gamma2, beta2):
    x = xyz[:, 0, :]
    y = xyz[:, 1, :]
    z = xyz[:, 2, :]

    nx, ny, nz, gidx = _fps(x, y, z)

    knn = _knn(nx[:, :, None], ny[:, :, None], nz[:, :, None], x, y, z)

    xyz_t = jnp.transpose(xyz, (0, 2, 1)).reshape(B * N, 3)
    pts_t = jnp.transpose(points, (0, 2, 1)).reshape(B * N, C_IN)
    table = jnp.concatenate(
        [xyz_t, pts_t, jnp.zeros((B * N, CP - CC), jnp.float32)], axis=1)

    knn3 = knn.reshape(NW, CPW, 128)
    fps3 = gidx.reshape(NW, FCW, 128)
    xg4, ng4 = _sc_gather(knn3, fps3, table)
    xg = xg4.reshape(NROWS, CP)
    ng = ng4.reshape(NCENT, CP)

    w1p = jnp.concatenate(
        [W0, jnp.zeros((64, CP - CC), jnp.float32)], axis=1).T
    y1, st1 = _pass1(xg, ng, w1p)
    y2, st2 = _mid(y1, st1, gamma0.reshape(1, 64), beta0.reshape(1, 64), W1.T, 64)
    y3, st3 = _mid(y2, st2, gamma1.reshape(1, 64), beta1.reshape(1, 64), W2.T, 128)
    new_points = _pool(y3.reshape(NCENT, K, 128), st3,
                       gamma2.reshape(1, 128), beta2.reshape(1, 128))

    new_xyz = jnp.stack([nx, ny, nz], axis=1)
    return (new_xyz, new_points)


# speedup vs baseline: 8.9254x; 1.0112x over previous
import functools

import jax
import jax.numpy as jnp
from jax import lax
from jax.experimental import pallas as pl
from jax.experimental.pallas import tpu as pltpu
from jax.experimental.pallas import tpu_sc as plsc

B = 8
N = 4096
S = 1024
K = 32
C_IN = 64
CC = 3 + C_IN
CP = 128
EPS = 1e-5
NROWS = B * S * K
NCENT = B * S


def _fps_body(x_ref, y_ref, z_ref, nx_ref, ny_ref, nz_ref, gi_ref):
    x = x_ref[...]
    y = y_ref[...]
    z = z_ref[...]
    lane_n = lax.broadcasted_iota(jnp.int32, (B, N), 1)
    lane_s = lax.broadcasted_iota(jnp.int32, (B, S), 1)
    boff = lax.broadcasted_iota(jnp.int32, (B, S), 0) * N

    cx0 = x[:, 0:1]
    cy0 = y[:, 0:1]
    cz0 = z[:, 0:1]
    zs = jnp.zeros((B, S), jnp.float32)
    nx0 = jnp.where(lane_s == 0, cx0, zs)
    ny0 = jnp.where(lane_s == 0, cy0, zs)
    nz0 = jnp.where(lane_s == 0, cz0, zs)
    gi0 = jnp.where(lane_s == 0, boff, jnp.zeros((B, S), jnp.int32))
    mind0 = jnp.full((B, N), 1e10, jnp.float32)

    def body(i, st):
        cx, cy, cz, mind, nx, ny, nz, gi = st
        dx = x - cx
        dy = y - cy
        dz = z - cz
        d = dx * dx + dy * dy + dz * dz
        mind = jnp.minimum(mind, d)
        m = jnp.max(mind, axis=1, keepdims=True)
        cand = jnp.where(mind == m, lane_n, N)
        idx = jnp.min(cand, axis=1, keepdims=True)
        hit = lane_n == idx
        fz = jnp.zeros((B, N), jnp.float32)
        cx = jnp.sum(jnp.where(hit, x, fz), axis=1, keepdims=True)
        cy = jnp.sum(jnp.where(hit, y, fz), axis=1, keepdims=True)
        cz = jnp.sum(jnp.where(hit, z, fz), axis=1, keepdims=True)
        here = lane_s == i
        nx = jnp.where(here, cx, nx)
        ny = jnp.where(here, cy, ny)
        nz = jnp.where(here, cz, nz)
        gi = jnp.where(here, idx + boff, gi)
        return (cx, cy, cz, mind, nx, ny, nz, gi)

    st = (cx0, cy0, cz0, mind0, nx0, ny0, nz0, gi0)
    st = lax.fori_loop(1, S, body, st)
    _, _, _, _, nx, ny, nz, gi = st
    nx_ref[...] = nx
    ny_ref[...] = ny
    nz_ref[...] = nz
    gi_ref[...] = gi


def _fps(x, y, z):
    return pl.pallas_call(
        _fps_body,
        out_shape=(
            jax.ShapeDtypeStruct((B, S), jnp.float32),
            jax.ShapeDtypeStruct((B, S), jnp.float32),
            jax.ShapeDtypeStruct((B, S), jnp.float32),
            jax.ShapeDtypeStruct((B, S), jnp.int32),
        ),
    )(x, y, z)


SBLK = 256


def _knn_body(nx_ref, ny_ref, nz_ref, x_ref, y_ref, z_ref, out_ref):
    b = pl.program_id(0)
    xs = nx_ref[0]
    ys = ny_ref[0]
    zs = nz_ref[0]
    xr = x_ref[0]
    yr = y_ref[0]
    zr = z_ref[0]
    def _b(v):
        return v.astype(jnp.bfloat16).astype(jnp.float32)

    a2 = xs * xs + ys * ys + zs * zs
    b2 = xr * xr + yr * yr + zr * zr
    ab = _b(xs) * _b(xr) + _b(ys) * _b(yr) + _b(zs) * _b(zr)
    d = (a2 + b2) - 2.0 * ab
    iota = lax.broadcasted_iota(jnp.int32, (SBLK, N), 1)
    kiota = lax.broadcasted_iota(jnp.int32, (SBLK, K), 1)
    boff = b * N

    def body(k, st):
        d, acc = st
        idx = jnp.argmin(d, axis=1).astype(jnp.int32)[:, None]
        acc = jnp.where(kiota == k, idx + boff, acc)
        d = jnp.where(iota == idx, jnp.float32(jnp.inf), d)
        return (d, acc)

    _, acc = lax.fori_loop(0, K, body, (d, jnp.zeros((SBLK, K), jnp.int32)))
    out_ref[0] = acc


def _knn(nxc, nyc, nzc, x, y, z):
    grid = (B, S // SBLK)
    x = x[:, None, :]
    y = y[:, None, :]
    z = z[:, None, :]
    cent_spec = pl.BlockSpec((1, SBLK, 1), lambda b, j: (b, j, 0))
    pt_spec = pl.BlockSpec((1, 1, N), lambda b, j: (b, 0, 0))
    return pl.pallas_call(
        _knn_body,
        grid=grid,
        in_specs=[cent_spec, cent_spec, cent_spec, pt_spec, pt_spec, pt_spec],
        out_specs=pl.BlockSpec((1, SBLK, K), lambda b, j: (b, j, 0)),
        out_shape=jax.ShapeDtypeStruct((B, S, K), jnp.int32),
    )(nxc, nyc, nzc, x, y, z)


NW = 32
RPW = NROWS // NW
CPW = RPW // 128
FPW = NCENT // NW
FCW = FPW // 128


def _sc_gather_body(knn_hbm, fps_hbm, tab_hbm, xg_hbm, ng_hbm,
                    idx_v, fidx_v, rows_v, sem):
    wid = lax.axis_index("s") * 2 + lax.axis_index("c")
    pltpu.sync_copy(knn_hbm.at[wid], idx_v)
    pltpu.sync_copy(fps_hbm.at[wid], fidx_v)

    def body(j, _):
        pltpu.async_copy(tab_hbm.at[idx_v.at[j]], rows_v, sem).wait()
        pltpu.sync_copy(rows_v, xg_hbm.at[wid, j])
        return 0

    lax.fori_loop(0, CPW, body, 0)

    def fbody(j, _):
        pltpu.async_copy(tab_hbm.at[fidx_v.at[j]], rows_v, sem).wait()
        pltpu.sync_copy(rows_v, ng_hbm.at[wid, j])
        return 0

    lax.fori_loop(0, FCW, fbody, 0)


def _sc_gather(knn3, fps3, table):
    mesh = plsc.VectorSubcoreMesh(core_axis_name="c", subcore_axis_name="s")
    f = functools.partial(
        pl.kernel,
        mesh=mesh,
        out_type=(
            jax.ShapeDtypeStruct((NW, CPW, 128, CP), jnp.float32),
            jax.ShapeDtypeStruct((NW, FCW, 128, CP), jnp.float32),
        ),
        scratch_types=[
            pltpu.VMEM((CPW, 128), jnp.int32),
            pltpu.VMEM((FCW, 128), jnp.int32),
            pltpu.VMEM((128, CP), jnp.float32),
            pltpu.SemaphoreType.DMA,
        ],
    )(_sc_gather_body)
    return f(knn3, fps3, table)


RBLK = 2048
CBLK = RBLK // K
NBLK = NROWS // RBLK


def _pass1_body(xg_ref, ng_ref, w_ref, y_ref, st_ref, acc):
    i = pl.program_id(0)

    @pl.when(i == 0)
    def _():
        acc[...] = jnp.zeros_like(acc)

    xv = xg_ref[...]
    cm = ng_ref[...]
    col = lax.broadcasted_iota(jnp.int32, (CBLK, CP), 1)
    cz = jnp.where(col < 3, cm, jnp.zeros_like(cm))
    w = w_ref[...]
    y = jnp.dot(xv, w, preferred_element_type=jnp.float32)
    corr = jnp.dot(cz, w, preferred_element_type=jnp.float32)
    y3 = y.reshape(CBLK, K, 64) - corr[:, None, :]
    y2 = y3.reshape(RBLK, 64)
    y_ref[...] = y2
    acc[0:1, :] += jnp.sum(y2, axis=0, keepdims=True)
    acc[1:2, :] += jnp.sum(y2 * y2, axis=0, keepdims=True)

    @pl.when(i == NBLK - 1)
    def _():
        st_ref[...] = acc[...]


def _pass1(xg, ng, w1p):
    return pl.pallas_call(
        _pass1_body,
        grid=(NBLK,),
        in_specs=[
            pl.BlockSpec((RBLK, CP), lambda i: (i, 0)),
            pl.BlockSpec((CBLK, CP), lambda i: (i, 0)),
            pl.BlockSpec((CP, 64), lambda i: (0, 0)),
        ],
        out_specs=(
            pl.BlockSpec((RBLK, 64), lambda i: (i, 0)),
            pl.BlockSpec((8, 64), lambda i: (0, 0)),
        ),
        out_shape=(
            jax.ShapeDtypeStruct((NROWS, 64), jnp.float32),
            jax.ShapeDtypeStruct((8, 64), jnp.float32),
        ),
        scratch_shapes=[pltpu.VMEM((8, 64), jnp.float32)],
    )(xg, ng, w1p)


def _mid_body(cout, pool, y_ref, st_ref, g_ref, b_ref, w_ref, z_ref, so_ref, acc):
    i = pl.program_id(0)

    @pl.when(i == 0)
    def _():
        acc[...] = jnp.zeros_like(acc)

    cnt = jnp.float32(NROWS)
    mean = st_ref[0:1, :] / cnt
    var = st_ref[1:2, :] / cnt - mean * mean
    scale = g_ref[...] / jnp.sqrt(var + EPS)
    shift = b_ref[...] - mean * scale
    y = y_ref[...]
    h = jnp.maximum(y * scale + shift, 0.0)
    z = jnp.dot(h, w_ref[...], preferred_element_type=jnp.float32)
    if pool:
        z_ref[...] = jnp.max(z.reshape(CBLK, K, cout), axis=1)
    else:
        z_ref[...] = z
    acc[0:1, :] += jnp.sum(z, axis=0, keepdims=True)
    acc[1:2, :] += jnp.sum(z * z, axis=0, keepdims=True)

    @pl.when(i == NBLK - 1)
    def _():
        so_ref[...] = acc[...]


def _mid(y, st, g, b, wt, cout, pool=False):
    orows = NCENT if pool else NROWS
    oblk = CBLK if pool else RBLK
    return pl.pallas_call(
        functools.partial(_mid_body, cout, pool),
        grid=(NBLK,),
        in_specs=[
            pl.BlockSpec((RBLK, 64), lambda i: (i, 0)),
            pl.BlockSpec((8, 64), lambda i: (0, 0)),
            pl.BlockSpec((1, 64), lambda i: (0, 0)),
            pl.BlockSpec((1, 64), lambda i: (0, 0)),
            pl.BlockSpec((64, cout), lambda i: (0, 0)),
        ],
        out_specs=(
            pl.BlockSpec((oblk, cout), lambda i: (i, 0)),
            pl.BlockSpec((8, cout), lambda i: (0, 0)),
        ),
        out_shape=(
            jax.ShapeDtypeStruct((orows, cout), jnp.float32),
            jax.ShapeDtypeStruct((8, cout), jnp.float32),
        ),
        scratch_shapes=[pltpu.VMEM((8, cout), jnp.float32)],
    )(y, st, g, b, wt)


def _pool_body(y_ref, st_ref, g_ref, b_ref, o_ref):
    i = pl.program_id(0)
    cnt = jnp.float32(NROWS)
    mean = st_ref[0:1, :] / cnt
    var = st_ref[1:2, :] / cnt - mean * mean
    scale = g_ref[...] / jnp.sqrt(var + EPS)
    shift = b_ref[...] - mean * scale
    y = y_ref[...]
    h = jnp.maximum(y * scale + shift, 0.0)
    o_ref[0, :, pl.ds((i % 8) * 128, 128)] = h.T


def _pool(zmax, st, g, b):
    return pl.pallas_call(
        _pool_body,
        grid=(NCENT // 128,),
        in_specs=[
            pl.BlockSpec((128, 128), lambda i: (i, 0)),
            pl.BlockSpec((8, 128), lambda i: (0, 0)),
            pl.BlockSpec((1, 128), lambda i: (0, 0)),
            pl.BlockSpec((1, 128), lambda i: (0, 0)),
        ],
        out_specs=pl.BlockSpec((1, 128, S), lambda i: (i // 8, 0, 0)),
        out_shape=jax.ShapeDtypeStruct((B, 128, S), jnp.float32),
    )(zmax, st, g, b)


def kernel(xyz, points, W0, gamma0, beta0, W1, gamma1, beta1, W2, gamma2, beta2):
    x = xyz[:, 0, :]
    y = xyz[:, 1, :]
    z = xyz[:, 2, :]

    nx, ny, nz, gidx = _fps(x, y, z)

    knn = _knn(nx[:, :, None], ny[:, :, None], nz[:, :, None], x, y, z)

    xyz_t = jnp.transpose(xyz, (0, 2, 1)).reshape(B * N, 3)
    pts_t = jnp.transpose(points, (0, 2, 1)).reshape(B * N, C_IN)
    table = jnp.concatenate(
        [xyz_t, pts_t, jnp.zeros((B * N, CP - CC), jnp.float32)], axis=1)

    knn3 = knn.reshape(NW, CPW, 128)
    fps3 = gidx.reshape(NW, FCW, 128)
    xg4, ng4 = _sc_gather(knn3, fps3, table)
    xg = xg4.reshape(NROWS, CP)
    ng = ng4.reshape(NCENT, CP)

    w1p = jnp.concatenate(
        [W0, jnp.zeros((64, CP - CC), jnp.float32)], axis=1).T
    y1, st1 = _pass1(xg, ng, w1p)
    y2, st2 = _mid(y1, st1, gamma0.reshape(1, 64), beta0.reshape(1, 64), W1.T, 64)
    zmax, st3 = _mid(y2, st2, gamma1.reshape(1, 64), beta1.reshape(1, 64), W2.T,
                     128, pool=True)
    new_points = _pool(zmax, st3, gamma2.reshape(1, 128), beta2.reshape(1, 128))

    new_xyz = jnp.stack([nx, ny, nz], axis=1)
    return (new_xyz, new_points)


# speedup vs baseline: 9.4152x; 1.0549x over previous
import functools

import jax
import jax.numpy as jnp
from jax import lax
from jax.experimental import pallas as pl
from jax.experimental.pallas import tpu as pltpu
from jax.experimental.pallas import tpu_sc as plsc

B = 8
N = 4096
S = 1024
K = 32
C_IN = 64
CC = 3 + C_IN
CP = 128
EPS = 1e-5
NROWS = B * S * K
NCENT = B * S


def _fps_body(x_ref, y_ref, z_ref, nx_ref, ny_ref, nz_ref, gi_ref):
    x = x_ref[...]
    y = y_ref[...]
    z = z_ref[...]
    lane_n = lax.broadcasted_iota(jnp.int32, (B, N), 1)
    lane_s = lax.broadcasted_iota(jnp.int32, (B, S), 1)
    boff = lax.broadcasted_iota(jnp.int32, (B, S), 0) * N

    cx0 = x[:, 0:1]
    cy0 = y[:, 0:1]
    cz0 = z[:, 0:1]
    zs = jnp.zeros((B, S), jnp.float32)
    nx0 = jnp.where(lane_s == 0, cx0, zs)
    ny0 = jnp.where(lane_s == 0, cy0, zs)
    nz0 = jnp.where(lane_s == 0, cz0, zs)
    gi0 = jnp.where(lane_s == 0, boff, jnp.zeros((B, S), jnp.int32))
    mind0 = jnp.full((B, N), 1e10, jnp.float32)

    def body(i, st):
        cx, cy, cz, mind, nx, ny, nz, gi = st
        dx = x - cx
        dy = y - cy
        dz = z - cz
        d = dx * dx + dy * dy + dz * dz
        mind = jnp.minimum(mind, d)
        m = jnp.max(mind, axis=1, keepdims=True)
        cand = jnp.where(mind == m, lane_n, N)
        idx = jnp.min(cand, axis=1, keepdims=True)
        hit = lane_n == idx
        fz = jnp.zeros((B, N), jnp.float32)
        cx = jnp.sum(jnp.where(hit, x, fz), axis=1, keepdims=True)
        cy = jnp.sum(jnp.where(hit, y, fz), axis=1, keepdims=True)
        cz = jnp.sum(jnp.where(hit, z, fz), axis=1, keepdims=True)
        here = lane_s == i
        nx = jnp.where(here, cx, nx)
        ny = jnp.where(here, cy, ny)
        nz = jnp.where(here, cz, nz)
        gi = jnp.where(here, idx + boff, gi)
        return (cx, cy, cz, mind, nx, ny, nz, gi)

    st = (cx0, cy0, cz0, mind0, nx0, ny0, nz0, gi0)
    st = lax.fori_loop(1, S, body, st)
    _, _, _, _, nx, ny, nz, gi = st
    nx_ref[...] = nx
    ny_ref[...] = ny
    nz_ref[...] = nz
    gi_ref[...] = gi


def _fps(x, y, z):
    return pl.pallas_call(
        _fps_body,
        out_shape=(
            jax.ShapeDtypeStruct((B, S), jnp.float32),
            jax.ShapeDtypeStruct((B, S), jnp.float32),
            jax.ShapeDtypeStruct((B, S), jnp.float32),
            jax.ShapeDtypeStruct((B, S), jnp.int32),
        ),
    )(x, y, z)


SBLK = 512


def _knn_body(nx_ref, ny_ref, nz_ref, x_ref, y_ref, z_ref, out_ref):
    b = pl.program_id(0)
    xs = nx_ref[0]
    ys = ny_ref[0]
    zs = nz_ref[0]
    xr = x_ref[0]
    yr = y_ref[0]
    zr = z_ref[0]
    def _b(v):
        return v.astype(jnp.bfloat16).astype(jnp.float32)

    a2 = xs * xs + ys * ys + zs * zs
    b2 = xr * xr + yr * yr + zr * zr
    ab = _b(xs) * _b(xr) + _b(ys) * _b(yr) + _b(zs) * _b(zr)
    d = (a2 + b2) - 2.0 * ab
    iota = lax.broadcasted_iota(jnp.int32, (SBLK, N), 1)
    kiota = lax.broadcasted_iota(jnp.int32, (SBLK, K), 1)
    boff = b * N

    inf = jnp.float32(jnp.inf)

    def body(k, st):
        mprev, iprev, acc = st
        elig = (d > mprev) | ((d == mprev) & (iota > iprev))
        dd = jnp.where(elig, d, inf)
        m = jnp.min(dd, axis=1, keepdims=True)
        cand = jnp.where(dd == m, iota, N)
        idx = jnp.min(cand, axis=1, keepdims=True)
        acc = jnp.where(kiota == k, idx + boff, acc)
        return (m, idx, acc)

    st0 = (jnp.full((SBLK, 1), -inf), jnp.full((SBLK, 1), -1, jnp.int32),
           jnp.zeros((SBLK, K), jnp.int32))
    _, _, acc = lax.fori_loop(0, K, body, st0)
    out_ref[0] = acc


def _knn(nxc, nyc, nzc, x, y, z):
    grid = (B, S // SBLK)
    x = x[:, None, :]
    y = y[:, None, :]
    z = z[:, None, :]
    cent_spec = pl.BlockSpec((1, SBLK, 1), lambda b, j: (b, j, 0))
    pt_spec = pl.BlockSpec((1, 1, N), lambda b, j: (b, 0, 0))
    return pl.pallas_call(
        _knn_body,
        grid=grid,
        in_specs=[cent_spec, cent_spec, cent_spec, pt_spec, pt_spec, pt_spec],
        out_specs=pl.BlockSpec((1, SBLK, K), lambda b, j: (b, j, 0)),
        out_shape=jax.ShapeDtypeStruct((B, S, K), jnp.int32),
    )(nxc, nyc, nzc, x, y, z)


NW = 32
RPW = NROWS // NW
CPW = RPW // 128
FPW = NCENT // NW
FCW = FPW // 128


def _sc_gather_body(knn_hbm, fps_hbm, tab_hbm, xg_hbm, ng_hbm,
                    idx_v, fidx_v, rows_v, sem):
    wid = lax.axis_index("s") * 2 + lax.axis_index("c")
    pltpu.sync_copy(knn_hbm.at[wid], idx_v)
    pltpu.sync_copy(fps_hbm.at[wid], fidx_v)

    def body(j, _):
        pltpu.async_copy(tab_hbm.at[idx_v.at[j]], rows_v, sem).wait()
        pltpu.sync_copy(rows_v, xg_hbm.at[wid, j])
        return 0

    lax.fori_loop(0, CPW, body, 0)

    def fbody(j, _):
        pltpu.async_copy(tab_hbm.at[fidx_v.at[j]], rows_v, sem).wait()
        pltpu.sync_copy(rows_v, ng_hbm.at[wid, j])
        return 0

    lax.fori_loop(0, FCW, fbody, 0)


def _sc_gather(knn3, fps3, table):
    mesh = plsc.VectorSubcoreMesh(core_axis_name="c", subcore_axis_name="s")
    f = functools.partial(
        pl.kernel,
        mesh=mesh,
        out_type=(
            jax.ShapeDtypeStruct((NW, CPW, 128, CP), jnp.float32),
            jax.ShapeDtypeStruct((NW, FCW, 128, CP), jnp.float32),
        ),
        scratch_types=[
            pltpu.VMEM((CPW, 128), jnp.int32),
            pltpu.VMEM((FCW, 128), jnp.int32),
            pltpu.VMEM((128, CP), jnp.float32),
            pltpu.SemaphoreType.DMA,
        ],
    )(_sc_gather_body)
    return f(knn3, fps3, table)


RBLK = 2048
CBLK = RBLK // K
NBLK = NROWS // RBLK


def _pass1_body(xg_ref, ng_ref, w_ref, y_ref, st_ref, acc):
    i = pl.program_id(0)

    @pl.when(i == 0)
    def _():
        acc[...] = jnp.zeros_like(acc)

    xv = xg_ref[...]
    cm = ng_ref[...]
    col = lax.broadcasted_iota(jnp.int32, (CBLK, CP), 1)
    cz = jnp.where(col < 3, cm, jnp.zeros_like(cm))
    w = w_ref[...]
    y = jnp.dot(xv, w, preferred_element_type=jnp.float32)
    corr = jnp.dot(cz, w, preferred_element_type=jnp.float32)
    y3 = y.reshape(CBLK, K, 64) - corr[:, None, :]
    y2 = y3.reshape(RBLK, 64)
    y_ref[...] = y2
    acc[0:1, :] += jnp.sum(y2, axis=0, keepdims=True)
    acc[1:2, :] += jnp.sum(y2 * y2, axis=0, keepdims=True)

    @pl.when(i == NBLK - 1)
    def _():
        st_ref[...] = acc[...]


def _pass1(xg, ng, w1p):
    return pl.pallas_call(
        _pass1_body,
        grid=(NBLK,),
        in_specs=[
            pl.BlockSpec((RBLK, CP), lambda i: (i, 0)),
            pl.BlockSpec((CBLK, CP), lambda i: (i, 0)),
            pl.BlockSpec((CP, 64), lambda i: (0, 0)),
        ],
        out_specs=(
            pl.BlockSpec((RBLK, 64), lambda i: (i, 0)),
            pl.BlockSpec((8, 64), lambda i: (0, 0)),
        ),
        out_shape=(
            jax.ShapeDtypeStruct((NROWS, 64), jnp.float32),
            jax.ShapeDtypeStruct((8, 64), jnp.float32),
        ),
        scratch_shapes=[pltpu.VMEM((8, 64), jnp.float32)],
    )(xg, ng, w1p)


def _mid_body(cout, pool, y_ref, st_ref, g_ref, b_ref, w_ref, z_ref, so_ref, acc):
    i = pl.program_id(0)

    @pl.when(i == 0)
    def _():
        acc[...] = jnp.zeros_like(acc)

    cnt = jnp.float32(NROWS)
    mean = st_ref[0:1, :] / cnt
    var = st_ref[1:2, :] / cnt - mean * mean
    scale = g_ref[...] / jnp.sqrt(var + EPS)
    shift = b_ref[...] - mean * scale
    y = y_ref[...]
    h = jnp.maximum(y * scale + shift, 0.0)
    z = jnp.dot(h, w_ref[...], preferred_element_type=jnp.float32)
    if pool:
        z_ref[...] = jnp.max(z.reshape(CBLK, K, cout), axis=1)
    else:
        z_ref[...] = z
    acc[0:1, :] += jnp.sum(z, axis=0, keepdims=True)
    acc[1:2, :] += jnp.sum(z * z, axis=0, keepdims=True)

    @pl.when(i == NBLK - 1)
    def _():
        so_ref[...] = acc[...]


def _mid(y, st, g, b, wt, cout, pool=False):
    orows = NCENT if pool else NROWS
    oblk = CBLK if pool else RBLK
    return pl.pallas_call(
        functools.partial(_mid_body, cout, pool),
        grid=(NBLK,),
        in_specs=[
            pl.BlockSpec((RBLK, 64), lambda i: (i, 0)),
            pl.BlockSpec((8, 64), lambda i: (0, 0)),
            pl.BlockSpec((1, 64), lambda i: (0, 0)),
            pl.BlockSpec((1, 64), lambda i: (0, 0)),
            pl.BlockSpec((64, cout), lambda i: (0, 0)),
        ],
        out_specs=(
            pl.BlockSpec((oblk, cout), lambda i: (i, 0)),
            pl.BlockSpec((8, cout), lambda i: (0, 0)),
        ),
        out_shape=(
            jax.ShapeDtypeStruct((orows, cout), jnp.float32),
            jax.ShapeDtypeStruct((8, cout), jnp.float32),
        ),
        scratch_shapes=[pltpu.VMEM((8, cout), jnp.float32)],
    )(y, st, g, b, wt)


def _pool_body(y_ref, st_ref, g_ref, b_ref, o_ref):
    i = pl.program_id(0)
    cnt = jnp.float32(NROWS)
    mean = st_ref[0:1, :] / cnt
    var = st_ref[1:2, :] / cnt - mean * mean
    scale = g_ref[...] / jnp.sqrt(var + EPS)
    shift = b_ref[...] - mean * scale
    y = y_ref[...]
    h = jnp.maximum(y * scale + shift, 0.0)
    o_ref[0, :, pl.ds((i % 8) * 128, 128)] = h.T


def _pool(zmax, st, g, b):
    return pl.pallas_call(
        _pool_body,
        grid=(NCENT // 128,),
        in_specs=[
            pl.BlockSpec((128, 128), lambda i: (i, 0)),
            pl.BlockSpec((8, 128), lambda i: (0, 0)),
            pl.BlockSpec((1, 128), lambda i: (0, 0)),
            pl.BlockSpec((1, 128), lambda i: (0, 0)),
        ],
        out_specs=pl.BlockSpec((1, 128, S), lambda i: (i // 8, 0, 0)),
        out_shape=jax.ShapeDtypeStruct((B, 128, S), jnp.float32),
    )(zmax, st, g, b)


def kernel(xyz, points, W0, gamma0, beta0, W1, gamma1, beta1, W2, gamma2, beta2):
    x = xyz[:, 0, :]
    y = xyz[:, 1, :]
    z = xyz[:, 2, :]

    nx, ny, nz, gidx = _fps(x, y, z)

    knn = _knn(nx[:, :, None], ny[:, :, None], nz[:, :, None], x, y, z)

    xyz_t = jnp.transpose(xyz, (0, 2, 1)).reshape(B * N, 3)
    pts_t = jnp.transpose(points, (0, 2, 1)).reshape(B * N, C_IN)
    table = jnp.concatenate(
        [xyz_t, pts_t, jnp.zeros((B * N, CP - CC), jnp.float32)], axis=1)

    knn3 = knn.reshape(NW, CPW, 128)
    fps3 = gidx.reshape(NW, FCW, 128)
    xg4, ng4 = _sc_gather(knn3, fps3, table)
    xg = xg4.reshape(NROWS, CP)
    ng = ng4.reshape(NCENT, CP)

    w1p = jnp.concatenate(
        [W0, jnp.zeros((64, CP - CC), jnp.float32)], axis=1).T
    y1, st1 = _pass1(xg, ng, w1p)
    y2, st2 = _mid(y1, st1, gamma0.reshape(1, 64), beta0.reshape(1, 64), W1.T, 64)
    zmax, st3 = _mid(y2, st2, gamma1.reshape(1, 64), beta1.reshape(1, 64), W2.T,
                     128, pool=True)
    new_points = _pool(zmax, st3, gamma2.reshape(1, 128), beta2.reshape(1, 128))

    new_xyz = jnp.stack([nx, ny, nz], axis=1)
    return (new_xyz, new_points)


# speedup vs baseline: 9.5312x; 1.0123x over previous
import functools

import jax
import jax.numpy as jnp
from jax import lax
from jax.experimental import pallas as pl
from jax.experimental.pallas import tpu as pltpu
from jax.experimental.pallas import tpu_sc as plsc

B = 8
N = 4096
S = 1024
K = 32
C_IN = 64
CC = 3 + C_IN
CP = 128
EPS = 1e-5
NROWS = B * S * K
NCENT = B * S


def _fps_body(x_ref, y_ref, z_ref, nx_ref, ny_ref, nz_ref, gi_ref):
    x = x_ref[...]
    y = y_ref[...]
    z = z_ref[...]
    lane_n = lax.broadcasted_iota(jnp.int32, (B, N), 1)
    lane_s = lax.broadcasted_iota(jnp.int32, (B, S), 1)
    boff = lax.broadcasted_iota(jnp.int32, (B, S), 0) * N

    cx0 = x[:, 0:1]
    cy0 = y[:, 0:1]
    cz0 = z[:, 0:1]
    zs = jnp.zeros((B, S), jnp.float32)
    nx0 = jnp.where(lane_s == 0, cx0, zs)
    ny0 = jnp.where(lane_s == 0, cy0, zs)
    nz0 = jnp.where(lane_s == 0, cz0, zs)
    gi0 = jnp.where(lane_s == 0, boff, jnp.zeros((B, S), jnp.int32))
    mind0 = jnp.full((B, N), 1e10, jnp.float32)

    def body(i, st):
        cx, cy, cz, mind, nx, ny, nz, gi = st
        dx = x - cx
        dy = y - cy
        dz = z - cz
        d = dx * dx + dy * dy + dz * dz
        mind = jnp.minimum(mind, d)
        m = jnp.max(mind, axis=1, keepdims=True)
        cand = jnp.where(mind == m, lane_n, N)
        idx = jnp.min(cand, axis=1, keepdims=True)
        hit = lane_n == idx
        fz = jnp.zeros((B, N), jnp.float32)
        cx = jnp.sum(jnp.where(hit, x, fz), axis=1, keepdims=True)
        cy = jnp.sum(jnp.where(hit, y, fz), axis=1, keepdims=True)
        cz = jnp.sum(jnp.where(hit, z, fz), axis=1, keepdims=True)
        here = lane_s == i
        nx = jnp.where(here, cx, nx)
        ny = jnp.where(here, cy, ny)
        nz = jnp.where(here, cz, nz)
        gi = jnp.where(here, idx + boff, gi)
        return (cx, cy, cz, mind, nx, ny, nz, gi)

    st = (cx0, cy0, cz0, mind0, nx0, ny0, nz0, gi0)
    st = lax.fori_loop(1, S, body, st)
    _, _, _, _, nx, ny, nz, gi = st
    nx_ref[...] = nx
    ny_ref[...] = ny
    nz_ref[...] = nz
    gi_ref[...] = gi


def _fps(x, y, z):
    return pl.pallas_call(
        _fps_body,
        out_shape=(
            jax.ShapeDtypeStruct((B, S), jnp.float32),
            jax.ShapeDtypeStruct((B, S), jnp.float32),
            jax.ShapeDtypeStruct((B, S), jnp.float32),
            jax.ShapeDtypeStruct((B, S), jnp.int32),
        ),
    )(x, y, z)


SBLK = 512


def _knn_body(nx_ref, ny_ref, nz_ref, x_ref, y_ref, z_ref, out_ref):
    b = pl.program_id(0)
    xs = nx_ref[0]
    ys = ny_ref[0]
    zs = nz_ref[0]
    xr = x_ref[0]
    yr = y_ref[0]
    zr = z_ref[0]
    def _b(v):
        return v.astype(jnp.bfloat16).astype(jnp.float32)

    a2 = xs * xs + ys * ys + zs * zs
    b2 = xr * xr + yr * yr + zr * zr
    ab = _b(xs) * _b(xr) + _b(ys) * _b(yr) + _b(zs) * _b(zr)
    d = (a2 + b2) - 2.0 * ab
    iota = lax.broadcasted_iota(jnp.int32, (SBLK, N), 1)
    kiota = lax.broadcasted_iota(jnp.int32, (SBLK, K), 1)
    boff = b * N

    inf = jnp.float32(jnp.inf)

    def body(k, st):
        mprev, iprev, acc = st
        elig = (d > mprev) | ((d == mprev) & (iota > iprev))
        dd = jnp.where(elig, d, inf)
        m = jnp.min(dd, axis=1, keepdims=True)
        cand = jnp.where(dd == m, iota, N)
        idx = jnp.min(cand, axis=1, keepdims=True)
        acc = jnp.where(kiota == k, idx + boff, acc)
        return (m, idx, acc)

    st0 = (jnp.full((SBLK, 1), -inf), jnp.full((SBLK, 1), -1, jnp.int32),
           jnp.zeros((SBLK, K), jnp.int32))
    _, _, acc = lax.fori_loop(0, K, body, st0)
    out_ref[0] = acc


def _knn(nxc, nyc, nzc, x, y, z):
    grid = (B, S // SBLK)
    x = x[:, None, :]
    y = y[:, None, :]
    z = z[:, None, :]
    cent_spec = pl.BlockSpec((1, SBLK, 1), lambda b, j: (b, j, 0))
    pt_spec = pl.BlockSpec((1, 1, N), lambda b, j: (b, 0, 0))
    return pl.pallas_call(
        _knn_body,
        grid=grid,
        in_specs=[cent_spec, cent_spec, cent_spec, pt_spec, pt_spec, pt_spec],
        out_specs=pl.BlockSpec((1, SBLK, K), lambda b, j: (b, j, 0)),
        out_shape=jax.ShapeDtypeStruct((B, S, K), jnp.int32),
    )(nxc, nyc, nzc, x, y, z)


NW = 32
RPW = NROWS // NW
CPW = RPW // 128
FPW = NCENT // NW
FCW = FPW // 128


def _sc_gather_body(knn_hbm, fps_hbm, tab_hbm, xg_hbm, ng_hbm,
                    idx_v, fidx_v, r0, r1, r2, r3, s0, s1, s2, s3):
    wid = lax.axis_index("s") * 2 + lax.axis_index("c")
    pltpu.sync_copy(knn_hbm.at[wid], idx_v)
    pltpu.sync_copy(fps_hbm.at[wid], fidx_v)
    bufs = (r0, r1, r2, r3)
    sems = (s0, s1, s2, s3)

    def body(j4, _):
        base = j4 * 4
        cps = [pltpu.async_copy(tab_hbm.at[idx_v.at[base + t]], bufs[t], sems[t])
               for t in range(4)]
        for t in range(4):
            cps[t].wait()
            pltpu.sync_copy(bufs[t], xg_hbm.at[wid, base + t])
        return 0

    lax.fori_loop(0, CPW // 4, body, 0)

    def fbody(j, _):
        pltpu.async_copy(tab_hbm.at[fidx_v.at[j]], r0, s0).wait()
        pltpu.sync_copy(r0, ng_hbm.at[wid, j])
        return 0

    lax.fori_loop(0, FCW, fbody, 0)


def _sc_gather(knn3, fps3, table):
    mesh = plsc.VectorSubcoreMesh(core_axis_name="c", subcore_axis_name="s")
    f = functools.partial(
        pl.kernel,
        mesh=mesh,
        out_type=(
            jax.ShapeDtypeStruct((NW, CPW, 128, CP), jnp.float32),
            jax.ShapeDtypeStruct((NW, FCW, 128, CP), jnp.float32),
        ),
        scratch_types=[
            pltpu.VMEM((CPW, 128), jnp.int32),
            pltpu.VMEM((FCW, 128), jnp.int32),
            pltpu.VMEM((128, CP), jnp.float32),
            pltpu.VMEM((128, CP), jnp.float32),
            pltpu.VMEM((128, CP), jnp.float32),
            pltpu.VMEM((128, CP), jnp.float32),
            pltpu.SemaphoreType.DMA,
            pltpu.SemaphoreType.DMA,
            pltpu.SemaphoreType.DMA,
            pltpu.SemaphoreType.DMA,
        ],
    )(_sc_gather_body)
    return f(knn3, fps3, table)


RBLK = 2048
CBLK = RBLK // K
NBLK = NROWS // RBLK


def _pass1_body(xg_ref, ng_ref, w_ref, y_ref, st_ref, acc):
    i = pl.program_id(0)

    @pl.when(i == 0)
    def _():
        acc[...] = jnp.zeros_like(acc)

    xv = xg_ref[...]
    cm = ng_ref[...]
    col = lax.broadcasted_iota(jnp.int32, (CBLK, CP), 1)
    cz = jnp.where(col < 3, cm, jnp.zeros_like(cm))
    w = w_ref[...]
    y = jnp.dot(xv, w, preferred_element_type=jnp.float32)
    corr = jnp.dot(cz, w, preferred_element_type=jnp.float32)
    y3 = y.reshape(CBLK, K, 64) - corr[:, None, :]
    y2 = y3.reshape(RBLK, 64)
    y_ref[...] = y2
    acc[0:1, :] += jnp.sum(y2, axis=0, keepdims=True)
    acc[1:2, :] += jnp.sum(y2 * y2, axis=0, keepdims=True)

    @pl.when(i == NBLK - 1)
    def _():
        st_ref[...] = acc[...]


def _pass1(xg, ng, w1p):
    return pl.pallas_call(
        _pass1_body,
        grid=(NBLK,),
        in_specs=[
            pl.BlockSpec((RBLK, CP), lambda i: (i, 0)),
            pl.BlockSpec((CBLK, CP), lambda i: (i, 0)),
            pl.BlockSpec((CP, 64), lambda i: (0, 0)),
        ],
        out_specs=(
            pl.BlockSpec((RBLK, 64), lambda i: (i, 0)),
            pl.BlockSpec((8, 64), lambda i: (0, 0)),
        ),
        out_shape=(
            jax.ShapeDtypeStruct((NROWS, 64), jnp.float32),
            jax.ShapeDtypeStruct((8, 64), jnp.float32),
        ),
        scratch_shapes=[pltpu.VMEM((8, 64), jnp.float32)],
    )(xg, ng, w1p)


def _mid_body(cout, pool, y_ref, st_ref, g_ref, b_ref, w_ref, z_ref, so_ref, acc):
    i = pl.program_id(0)

    @pl.when(i == 0)
    def _():
        acc[...] = jnp.zeros_like(acc)

    cnt = jnp.float32(NROWS)
    mean = st_ref[0:1, :] / cnt
    var = st_ref[1:2, :] / cnt - mean * mean
    scale = g_ref[...] / jnp.sqrt(var + EPS)
    shift = b_ref[...] - mean * scale
    y = y_ref[...]
    h = jnp.maximum(y * scale + shift, 0.0)
    z = jnp.dot(h, w_ref[...], preferred_element_type=jnp.float32)
    if pool:
        z_ref[...] = jnp.max(z.reshape(CBLK, K, cout), axis=1)
    else:
        z_ref[...] = z
    acc[0:1, :] += jnp.sum(z, axis=0, keepdims=True)
    acc[1:2, :] += jnp.sum(z * z, axis=0, keepdims=True)

    @pl.when(i == NBLK - 1)
    def _():
        so_ref[...] = acc[...]


def _mid(y, st, g, b, wt, cout, pool=False):
    orows = NCENT if pool else NROWS
    oblk = CBLK if pool else RBLK
    return pl.pallas_call(
        functools.partial(_mid_body, cout, pool),
        grid=(NBLK,),
        in_specs=[
            pl.BlockSpec((RBLK, 64), lambda i: (i, 0)),
            pl.BlockSpec((8, 64), lambda i: (0, 0)),
            pl.BlockSpec((1, 64), lambda i: (0, 0)),
            pl.BlockSpec((1, 64), lambda i: (0, 0)),
            pl.BlockSpec((64, cout), lambda i: (0, 0)),
        ],
        out_specs=(
            pl.BlockSpec((oblk, cout), lambda i: (i, 0)),
            pl.BlockSpec((8, cout), lambda i: (0, 0)),
        ),
        out_shape=(
            jax.ShapeDtypeStruct((orows, cout), jnp.float32),
            jax.ShapeDtypeStruct((8, cout), jnp.float32),
        ),
        scratch_shapes=[pltpu.VMEM((8, cout), jnp.float32)],
    )(y, st, g, b, wt)


def _pool_body(y_ref, st_ref, g_ref, b_ref, o_ref):
    i = pl.program_id(0)
    cnt = jnp.float32(NROWS)
    mean = st_ref[0:1, :] / cnt
    var = st_ref[1:2, :] / cnt - mean * mean
    scale = g_ref[...] / jnp.sqrt(var + EPS)
    shift = b_ref[...] - mean * scale
    y = y_ref[...]
    h = jnp.maximum(y * scale + shift, 0.0)
    o_ref[0, :, pl.ds((i % 8) * 128, 128)] = h.T


def _pool(zmax, st, g, b):
    return pl.pallas_call(
        _pool_body,
        grid=(NCENT // 128,),
        in_specs=[
            pl.BlockSpec((128, 128), lambda i: (i, 0)),
            pl.BlockSpec((8, 128), lambda i: (0, 0)),
            pl.BlockSpec((1, 128), lambda i: (0, 0)),
            pl.BlockSpec((1, 128), lambda i: (0, 0)),
        ],
        out_specs=pl.BlockSpec((1, 128, S), lambda i: (i // 8, 0, 0)),
        out_shape=jax.ShapeDtypeStruct((B, 128, S), jnp.float32),
    )(zmax, st, g, b)


def kernel(xyz, points, W0, gamma0, beta0, W1, gamma1, beta1, W2, gamma2, beta2):
    x = xyz[:, 0, :]
    y = xyz[:, 1, :]
    z = xyz[:, 2, :]

    nx, ny, nz, gidx = _fps(x, y, z)

    knn = _knn(nx[:, :, None], ny[:, :, None], nz[:, :, None], x, y, z)

    xyz_t = jnp.transpose(xyz, (0, 2, 1)).reshape(B * N, 3)
    pts_t = jnp.transpose(points, (0, 2, 1)).reshape(B * N, C_IN)
    table = jnp.concatenate(
        [xyz_t, pts_t, jnp.zeros((B * N, CP - CC), jnp.float32)], axis=1)

    knn3 = knn.reshape(NW, CPW, 128)
    fps3 = gidx.reshape(NW, FCW, 128)
    xg4, ng4 = _sc_gather(knn3, fps3, table)
    xg = xg4.reshape(NROWS, CP)
    ng = ng4.reshape(NCENT, CP)

    w1p = jnp.concatenate(
        [W0, jnp.zeros((64, CP - CC), jnp.float32)], axis=1).T
    y1, st1 = _pass1(xg, ng, w1p)
    y2, st2 = _mid(y1, st1, gamma0.reshape(1, 64), beta0.reshape(1, 64), W1.T, 64)
    zmax, st3 = _mid(y2, st2, gamma1.reshape(1, 64), beta1.reshape(1, 64), W2.T,
                     128, pool=True)
    new_points = _pool(zmax, st3, gamma2.reshape(1, 128), beta2.reshape(1, 128))

    new_xyz = jnp.stack([nx, ny, nz], axis=1)
    return (new_xyz, new_points)
